# Initial kernel scaffold; baseline (speedup 1.0000x reference)
import jax
import jax.numpy as jnp
from jax.experimental import pallas as pl


def kernel(feat1, feat2, edge_index1, edge_index2, W1, b1, W2, b2, Wm1, bm1, Wm2, bm2):
    raise NotImplementedError("write your pallas kernel here")


# speedup vs baseline: 3.2515x; 3.2515x over previous
import functools

import jax
import jax.numpy as jnp
from jax import lax
from jax.experimental import pallas as pl
from jax.experimental.pallas import tpu as pltpu
from jax.experimental.pallas import tpu_sc as plsc

N = 10000
E = 320000
D = 128

NC = 2
NS = 16
NW = NC * NS
EPW = E // NW
C = 80
NCHUNK = EPW // C
NPAD = 10240
RPT = NPAD // NS
DPT = NPAD // NS

_sc_mesh = plsc.VectorSubcoreMesh(core_axis_name="c", subcore_axis_name="s")


@functools.partial(
    pl.kernel,
    out_type=[
        jax.ShapeDtypeStruct((NC * NPAD,), jnp.float32),
        jax.ShapeDtypeStruct((NC * NPAD,), jnp.float32),
    ],
    mesh=_sc_mesh,
    scratch_types=[
        pltpu.VMEM((C,), jnp.int32),
        pltpu.VMEM((C,), jnp.int32),
        pltpu.VMEM((C,), jnp.float32),
        pltpu.VMEM_SHARED((NPAD,), jnp.float32),
        pltpu.VMEM_SHARED((NPAD,), jnp.float32),
    ],
)
def _sc_degrees(src_hbm, dst_hbm, ones_hbm, zeros_hbm, outo_hbm, outi_hbm,
                sidx_v, didx_v, ones_v, dego_sh, degi_sh):
    cid = lax.axis_index("c")
    sid = lax.axis_index("s")
    wid = cid * NS + sid
    pltpu.sync_copy(ones_hbm, ones_v)
    pltpu.sync_copy(zeros_hbm, dego_sh.at[pl.ds(sid * DPT, DPT)])
    pltpu.sync_copy(zeros_hbm, degi_sh.at[pl.ds(sid * DPT, DPT)])
    plsc.subcore_barrier()
    base = wid * EPW

    def body(i, carry):
        off = base + i * C
        pltpu.sync_copy(src_hbm.at[pl.ds(off, C)], sidx_v)
        pltpu.sync_copy(dst_hbm.at[pl.ds(off, C)], didx_v)
        pltpu.sync_copy(ones_v, dego_sh.at[sidx_v], add=True)
        pltpu.sync_copy(ones_v, degi_sh.at[didx_v], add=True)
        return carry

    lax.fori_loop(0, NCHUNK, body, 0)
    plsc.subcore_barrier()
    pltpu.sync_copy(dego_sh.at[pl.ds(sid * DPT, DPT)],
                    outo_hbm.at[pl.ds(cid * NPAD + sid * DPT, DPT)])
    pltpu.sync_copy(degi_sh.at[pl.ds(sid * DPT, DPT)],
                    outi_hbm.at[pl.ds(cid * NPAD + sid * DPT, DPT)])


@functools.partial(
    pl.kernel,
    out_type=jax.ShapeDtypeStruct((NC, NPAD, D), jnp.float32),
    mesh=_sc_mesh,
    scratch_types=[
        pltpu.VMEM((C,), jnp.int32),
        pltpu.VMEM((C,), jnp.int32),
        pltpu.VMEM((C, D), jnp.float32),
        pltpu.VMEM_SHARED((NPAD, D), jnp.float32),
        pltpu.SemaphoreType.DMA,
    ],
)
def _sc_spmm(h_hbm, src_hbm, dst_hbm, zrows_hbm, out_hbm,
             sidx_v, didx_v, rows_v, acc_sh, sem):
    cid = lax.axis_index("c")
    sid = lax.axis_index("s")
    wid = cid * NS + sid
    pltpu.sync_copy(zrows_hbm, acc_sh.at[pl.ds(sid * RPT, RPT)])
    plsc.subcore_barrier()
    base = wid * EPW

    def body(i, carry):
        off = base + i * C
        pltpu.sync_copy(src_hbm.at[pl.ds(off, C)], sidx_v)
        pltpu.sync_copy(dst_hbm.at[pl.ds(off, C)], didx_v)
        pltpu.async_copy(h_hbm.at[sidx_v], rows_v, sem).wait()
        pltpu.sync_copy(rows_v, acc_sh.at[didx_v], add=True)
        return carry

    lax.fori_loop(0, NCHUNK, body, 0)
    plsc.subcore_barrier()
    pltpu.sync_copy(acc_sh.at[pl.ds(sid * RPT, RPT)],
                    out_hbm.at[cid, pl.ds(sid * RPT, RPT)])


_BN = 2000


def _prep_body(feat_ref, doa_ref, dob_ref, dia_ref, dib_ref,
               h0_ref, ns_ref, nd_ref):
    dego = doa_ref[...] + dob_ref[...]
    degi = dia_ref[...] + dib_ref[...]
    ns = lax.rsqrt(jnp.where(dego > 0, dego, 1.0))
    nd = lax.rsqrt(jnp.where(degi > 0, degi, 1.0))
    ns_ref[...] = ns
    nd_ref[...] = nd
    h0_ref[...] = feat_ref[...] * ns


def _tc_prep(feat, doa, dob, dia, dib):
    grid = (N // _BN,)
    row = pl.BlockSpec((_BN, D), lambda i: (i, 0))
    col = pl.BlockSpec((_BN, 1), lambda i: (i, 0))
    return pl.pallas_call(
        _prep_body,
        grid=grid,
        in_specs=[row, col, col, col, col],
        out_specs=[row, col, col],
        out_shape=[
            jax.ShapeDtypeStruct((N, D), jnp.float32),
            jax.ShapeDtypeStruct((N, 1), jnp.float32),
            jax.ShapeDtypeStruct((N, 1), jnp.float32),
        ],
    )(feat, doa, dob, dia, dib)


def _mm_body(aa_ref, ab_ref, nd_ref, so_ref, w_ref, b_ref, y_ref):
    x = (aa_ref[...] + ab_ref[...]) * nd_ref[...]
    y = jnp.dot(x, w_ref[...], preferred_element_type=jnp.float32)
    y = jnp.maximum(y + b_ref[...], 0.0)
    y_ref[...] = y * so_ref[...]


def _tc_mm(agg_a, agg_b, nd, so, w, b):
    grid = (N // _BN,)
    row = pl.BlockSpec((_BN, D), lambda i: (i, 0))
    col = pl.BlockSpec((_BN, 1), lambda i: (i, 0))
    full = pl.BlockSpec((D, D), lambda i: (0, 0))
    vec = pl.BlockSpec((1, D), lambda i: (0, 0))
    return pl.pallas_call(
        _mm_body,
        grid=grid,
        in_specs=[row, row, col, col, full, vec],
        out_specs=row,
        out_shape=jax.ShapeDtypeStruct((N, D), jnp.float32),
    )(agg_a, agg_b, nd, so, w, b)


def _mlp_body(c_ref, w1_ref, b1_ref, w2_ref, b2_ref, y_ref):
    t = jnp.dot(c_ref[...], w1_ref[...], preferred_element_type=jnp.float32)
    t = jnp.maximum(t + b1_ref[...], 0.0)
    y = jnp.dot(t, w2_ref[...], preferred_element_type=jnp.float32)
    y_ref[...] = jnp.maximum(y + b2_ref[...], 0.0)


def _tc_mlp(c, w1, b1, w2, b2):
    grid = (N // _BN,)
    row = pl.BlockSpec((_BN, D), lambda i: (i, 0))
    full = pl.BlockSpec((D, D), lambda i: (0, 0))
    vec = pl.BlockSpec((1, D), lambda i: (0, 0))
    return pl.pallas_call(
        _mlp_body,
        grid=grid,
        in_specs=[row, full, vec, full, vec],
        out_specs=row,
        out_shape=jax.ShapeDtypeStruct((N, D), jnp.float32),
    )(c, w1, b1, w2, b2)


def _gcn_encode(feat, src, dst, W1, b1, W2, b2):
    ones_c = jnp.ones((C,), jnp.float32)
    zeros_deg = jnp.zeros((DPT,), jnp.float32)
    zrows = jnp.zeros((RPT, D), jnp.float32)

    dego, degi = _sc_degrees(src, dst, ones_c, zeros_deg)
    doa = dego[:N].reshape(N, 1)
    dob = dego[NPAD:NPAD + N].reshape(N, 1)
    dia = degi[:N].reshape(N, 1)
    dib = degi[NPAD:NPAD + N].reshape(N, 1)

    h0, ns, nd = _tc_prep(feat, doa, dob, dia, dib)

    agg1 = _sc_spmm(h0, src, dst, zrows)
    h1 = _tc_mm(agg1[0, :N], agg1[1, :N], nd, ns, W1, b1.reshape(1, D))

    agg2 = _sc_spmm(h1, src, dst, zrows)
    ones_n = jnp.ones((N, 1), jnp.float32)
    c = _tc_mm(agg2[0, :N], agg2[1, :N], nd, ones_n, W2, b2.reshape(1, D))
    return c


def kernel(feat1, feat2, edge_index1, edge_index2, W1, b1, W2, b2,
           Wm1, bm1, Wm2, bm2):
    s1, d1 = edge_index1[0], edge_index1[1]
    s2, d2 = edge_index2[0], edge_index2[1]
    c1 = _gcn_encode(feat1, s1, d1, W1, b1, W2, b2)
    c2 = _gcn_encode(feat2, s2, d2, W1, b1, W2, b2)
    x21 = _tc_mlp(c1, Wm1, bm1.reshape(1, D), Wm2, bm2.reshape(1, D))
    x22 = _tc_mlp(c2, Wm1, bm1.reshape(1, D), Wm2, bm2.reshape(1, D))
    return (c1, c2, x21, x22)
